# baseline (speedup 1.0000x reference)
import functools

import jax
import jax.numpy as jnp
from jax import lax
from jax.experimental import pallas as pl
from jax.experimental.pallas import tpu as pltpu
from jax.experimental.pallas import tpu_sc as plsc

VOCAB = 100000
MAXLEN = 2048
EMBED = 64
BATCH = 64
SEQ = 2048

NUM_CORES = 2
NUM_SUBCORES = 16
NW = NUM_CORES * NUM_SUBCORES
ROWS = BATCH * SEQ
BPW = ROWS // NW
CH = 512
NCH = BPW // CH
LANES = 16


def _make_kernel():
    mesh = plsc.VectorSubcoreMesh(core_axis_name="c", subcore_axis_name="s")

    @functools.partial(
        pl.kernel,
        mesh=mesh,
        out_type=jax.ShapeDtypeStruct((ROWS, EMBED), jnp.float32),
        compiler_params=pltpu.CompilerParams(use_tc_tiling_on_sc=False),
        scratch_types=[
            pltpu.VMEM((CH,), jnp.int32),
            pltpu.VMEM((CH, EMBED), jnp.float32),
            pltpu.VMEM((CH, EMBED), jnp.float32),
            pltpu.SemaphoreType.DMA,
        ],
    )
    def emb(x_hbm, tok_hbm, pos_hbm, out_hbm, idx_v, rows_v, pos_v, sem):
        c = lax.axis_index("c")
        s = lax.axis_index("s")
        wid = s * NUM_CORES + c
        base = wid * BPW

        def chunk(i, carry):
            off = base + i * CH
            pltpu.sync_copy(x_hbm.at[pl.ds(off, CH)], idx_v)
            gather = pltpu.async_copy(tok_hbm.at[idx_v], rows_v, sem)
            pos_start = (i * CH) % SEQ
            pltpu.sync_copy(pos_hbm.at[pl.ds(pos_start, CH)], pos_v)
            gather.wait()

            def rowfn(r, carry2):
                for cc in range(EMBED // LANES):
                    sl = pl.ds(cc * LANES, LANES)
                    rows_v[r, sl] = rows_v[r, sl] + pos_v[r, sl]
                return carry2

            lax.fori_loop(0, CH, rowfn, 0, unroll=4)
            pltpu.sync_copy(rows_v, out_hbm.at[pl.ds(off, CH)])
            return carry

        lax.fori_loop(0, NCH, chunk, 0)

    return emb


_emb = _make_kernel()


def kernel(x, token_table, pos_table):
    x_flat = x.reshape(ROWS).astype(jnp.int32)
    out = _emb(x_flat, token_table, pos_table)
    return out.reshape(BATCH, SEQ, EMBED)


# speedup vs baseline: 1.4423x; 1.4423x over previous
import functools

import jax
import jax.numpy as jnp
from jax import lax
from jax.experimental import pallas as pl
from jax.experimental.pallas import tpu as pltpu
from jax.experimental.pallas import tpu_sc as plsc

VOCAB = 100000
MAXLEN = 2048
EMBED = 64
BATCH = 64
SEQ = 2048

NUM_CORES = 2
NUM_SUBCORES = 16
NW = NUM_CORES * NUM_SUBCORES
NB = BATCH // (NW // 2)
HALF = SEQ // 2
CH = 256
CPB = HALF // CH
NCHUNK = NB * CPB
NBUF = 3
LANES = 16


def _make_kernel():
    mesh = plsc.VectorSubcoreMesh(core_axis_name="c", subcore_axis_name="s")

    @functools.partial(
        pl.kernel,
        mesh=mesh,
        out_type=jax.ShapeDtypeStruct((BATCH, SEQ, EMBED), jnp.float32),
        compiler_params=pltpu.CompilerParams(use_tc_tiling_on_sc=False),
        scratch_types=[
            pltpu.VMEM((NB, HALF), jnp.int32),
            pltpu.VMEM((HALF, EMBED), jnp.float32),
            pltpu.VMEM((NBUF, CH, EMBED), jnp.float32),
        ]
        + [pltpu.SemaphoreType.DMA] * (2 * NBUF + 1),
    )
    def emb(x_hbm, tok_hbm, pos_hbm, out_hbm, idx_v, pos_v, rows_v, *sems):
        gsem = sems[0:NBUF]
        ssem = sems[NBUF : 2 * NBUF]
        psem = sems[2 * NBUF]
        c = lax.axis_index("c")
        s = lax.axis_index("s")
        wid = s * NUM_CORES + c
        h = wid % 2
        bb = (wid // 2) * NB
        soff = h * HALF

        pos_cp = pltpu.async_copy(pos_hbm.at[pl.ds(soff, HALF)], pos_v, psem)
        for j in range(NB):
            pltpu.sync_copy(x_hbm.at[bb + j, pl.ds(soff, HALF)], idx_v.at[j])

        gathers = {}
        stores = {}

        def fire_gather(ci):
            j, o = divmod(ci, CPB)
            gathers[ci] = pltpu.async_copy(
                tok_hbm.at[idx_v.at[j, pl.ds(o * CH, CH)]],
                rows_v.at[ci % NBUF],
                gsem[ci % NBUF],
            )

        def fire_store(ci):
            j, o = divmod(ci, CPB)
            stores[ci] = pltpu.async_copy(
                rows_v.at[ci % NBUF],
                out_hbm.at[bb + j, pl.ds(soff + o * CH, CH), :],
                ssem[ci % NBUF],
            )

        fire_gather(0)
        fire_gather(1)
        pos_cp.wait()

        for ci in range(NCHUNK):
            b = ci % NBUF
            gathers[ci].wait()
            o = (ci % CPB) * CH

            def rowfn(r, carry, _b=b, _o=o):
                for cc in range(EMBED // LANES):
                    sl = pl.ds(cc * LANES, LANES)
                    rows_v[_b, r, sl] = rows_v[_b, r, sl] + pos_v[_o + r, sl]
                return carry

            lax.fori_loop(0, CH, rowfn, 0, unroll=4)
            fire_store(ci)
            if ci + 2 < NCHUNK:
                if ci - 1 >= 0:
                    stores[ci - 1].wait()
                fire_gather(ci + 2)

        stores[NCHUNK - 3].wait()
        stores[NCHUNK - 2].wait()
        stores[NCHUNK - 1].wait()

    return emb


_emb = _make_kernel()


def kernel(x, token_table, pos_table):
    return _emb(x.astype(jnp.int32), token_table, pos_table)
